# baseline (speedup 1.0000x reference)
import functools

import jax
import jax.numpy as jnp
from jax import lax
from jax.experimental import pallas as pl
from jax.experimental.pallas import tpu as pltpu
from jax.experimental.pallas import tpu_sc as plsc

N = 10000
D = 128
NC = 2
NS = 16
NW = NC * NS
CHUNK = 128

NBUF = 2
IR = 4

E_RAW = 320000
EPW = ((E_RAW + NW * CHUNK * IR - 1) // (NW * CHUNK * IR)) * CHUNK * IR
C_CH = EPW // CHUNK
C_G = C_CH // IR
E_PAD = EPW * NW

RPT = (((N + 1) + NS * 16 - 1) // (NS * 16)) * 16
N_PAD = RPT * NS
NPB = RPT // CHUNK

RB = 2000
GRID = N // RB

_mesh = plsc.VectorSubcoreMesh(core_axis_name="c", subcore_axis_name="s")


@functools.partial(
    pl.kernel,
    out_type=jax.ShapeDtypeStruct((NC, N_PAD), jnp.float32),
    mesh=_mesh,
    scratch_types=[
        pltpu.VMEM((C_CH, CHUNK), jnp.int32),
        pltpu.VMEM((CHUNK,), jnp.float32),
        pltpu.VMEM((RPT,), jnp.float32),
        pltpu.VMEM_SHARED((N_PAD,), jnp.float32),
    ],
)
def _deg_kernel(dst_hbm, deg_hbm, idx_v, ones_v, zer_v, deg_sh):
    cid = lax.axis_index("c")
    sid = lax.axis_index("s")
    wid = sid * NC + cid

    z16 = jnp.zeros((16,), jnp.float32)
    o16 = jnp.ones((16,), jnp.float32)

    def zi(i, _):
        zer_v[pl.ds(i * 16, 16)] = z16
        return 0

    lax.fori_loop(0, RPT // 16, zi, 0)

    def oi(i, _):
        ones_v[pl.ds(i * 16, 16)] = o16
        return 0

    lax.fori_loop(0, CHUNK // 16, oi, 0)

    pltpu.sync_copy(zer_v, deg_sh.at[pl.ds(sid * RPT, RPT)])
    pltpu.sync_copy(dst_hbm.at[wid], idx_v)
    plsc.subcore_barrier()

    def step(j, _):
        pltpu.sync_copy(ones_v, deg_sh.at[idx_v.at[j]], add=True)
        return 0

    lax.fori_loop(0, C_CH, step, 0)
    plsc.subcore_barrier()
    pltpu.sync_copy(
        deg_sh.at[pl.ds(sid * RPT, RPT)],
        deg_hbm.at[cid, pl.ds(sid * RPT, RPT)],
    )


@functools.partial(
    pl.kernel,
    out_type=jax.ShapeDtypeStruct((NC, N_PAD, D), jnp.float32),
    mesh=_mesh,
    scratch_types=[
        [pltpu.VMEM((2, CHUNK), jnp.int32)] * IR,
        [pltpu.SemaphoreType.DMA] * IR,
        [pltpu.VMEM((CHUNK, D), jnp.float32)] * NBUF,
        [pltpu.SemaphoreType.DMA] * NBUF,
        pltpu.SemaphoreType.DMA,
        pltpu.VMEM_SHARED((N_PAD, D), jnp.float32),
    ],
)
def _msg_kernel(h_hbm, edges_hbm, out_hbm, iring, isem, rows, gsem, ssem,
                acc_sh):
    cid = lax.axis_index("c")
    sid = lax.axis_index("s")
    wid = sid * NC + cid

    z16 = jnp.zeros((16,), jnp.float32)

    def zr(i, _):
        for k in range(D // 16):
            rows[0][i, pl.ds(k * 16, 16)] = z16
        return 0

    lax.fori_loop(0, CHUNK, zr, 0)

    for k in range(NPB):
        pltpu.sync_copy(
            rows[0], acc_sh.at[pl.ds(sid * RPT + k * CHUNK, CHUNK)]
        )
    plsc.subcore_barrier()

    for r in range(IR):
        pltpu.async_copy(edges_hbm.at[wid, r], iring[r], isem[r])
    for b in range(NBUF):
        pltpu.make_async_copy(edges_hbm.at[wid, b], iring[b], isem[b]).wait()
        pltpu.async_copy(h_hbm.at[iring[b].at[0]], rows[b], gsem[b])

    def group(g, _):
        for u in range(IR):
            j = g * IR + u
            br = u % NBUF
            pltpu.make_async_copy(h_hbm.at[iring[u].at[0]], rows[br],
                                  gsem[br]).wait()
            pltpu.async_copy(rows[br], acc_sh.at[iring[u].at[1]], ssem,
                             add=True).wait()

            @pl.when(j + IR < C_CH)
            def _():
                pltpu.async_copy(edges_hbm.at[wid, j + IR], iring[u], isem[u])

            @pl.when(j + NBUF < C_CH)
            def _():
                u2 = (u + NBUF) % IR
                pltpu.make_async_copy(edges_hbm.at[wid, 0], iring[u2],
                                      isem[u2]).wait()
                pltpu.async_copy(h_hbm.at[iring[u2].at[0]], rows[br], gsem[br])
        return 0

    lax.fori_loop(0, C_G, group, 0)
    plsc.subcore_barrier()

    for k in range(NPB):
        sl = pl.ds(sid * RPT + k * CHUNK, CHUNK)
        pltpu.sync_copy(acc_sh.at[sl], out_hbm.at[cid, sl])


def _mm_body(d0_ref, d1_ref, x_ref, w_ref, h_ref):
    deg = d0_ref[...] + d1_ref[...] + 1.0
    dis = lax.rsqrt(deg)
    h = jnp.dot(x_ref[...], w_ref[...], preferred_element_type=jnp.float32)
    h_ref[...] = h * dis


def _fin_body(a0_ref, a1_ref, hp_ref, d0_ref, d1_ref, b_ref, o_ref):
    deg = d0_ref[...] + d1_ref[...] + 1.0
    dis = lax.rsqrt(deg)
    acc = a0_ref[0] + a1_ref[0] + hp_ref[...]
    o_ref[...] = acc * dis + b_ref[...]


@jax.jit
def kernel(x, edge_index, W, b):
    src = edge_index[0].astype(jnp.int32)
    dst = edge_index[1].astype(jnp.int32)
    e = src.shape[0]
    pad = E_PAD - e
    src_p = jnp.concatenate([src, jnp.zeros((pad,), jnp.int32)])
    dst_p = jnp.concatenate([dst, jnp.full((pad,), N, jnp.int32)])
    src3 = src_p.reshape(NW, C_CH, CHUNK)
    dst3 = dst_p.reshape(NW, C_CH, CHUNK)
    edges = jnp.stack([src3, dst3], axis=2)

    dega = _deg_kernel(dst3)
    d0 = dega[0, :N].reshape(N, 1)
    d1 = dega[1, :N].reshape(N, 1)

    hp = pl.pallas_call(
        _mm_body,
        grid=(GRID,),
        in_specs=[
            pl.BlockSpec((RB, 1), lambda i: (i, 0)),
            pl.BlockSpec((RB, 1), lambda i: (i, 0)),
            pl.BlockSpec((RB, D), lambda i: (i, 0)),
            pl.BlockSpec((D, D), lambda i: (0, 0)),
        ],
        out_specs=pl.BlockSpec((RB, D), lambda i: (i, 0)),
        out_shape=jax.ShapeDtypeStruct((N, D), jnp.float32),
    )(d0, d1, x, W)

    acc = _msg_kernel(hp, edges)

    out = pl.pallas_call(
        _fin_body,
        grid=(GRID,),
        in_specs=[
            pl.BlockSpec((1, RB, D), lambda i: (0, i, 0)),
            pl.BlockSpec((1, RB, D), lambda i: (1, i, 0)),
            pl.BlockSpec((RB, D), lambda i: (i, 0)),
            pl.BlockSpec((RB, 1), lambda i: (i, 0)),
            pl.BlockSpec((RB, 1), lambda i: (i, 0)),
            pl.BlockSpec((1, D), lambda i: (0, 0)),
        ],
        out_specs=pl.BlockSpec((RB, D), lambda i: (i, 0)),
        out_shape=jax.ShapeDtypeStruct((N, D), jnp.float32),
    )(acc, acc, hp, d0, d1, b.reshape(1, D))
    return out
